# baseline (speedup 1.0000x reference)
import functools

import jax
import jax.numpy as jnp
from jax import lax
from jax.experimental import pallas as pl
from jax.experimental.pallas import tpu as pltpu
from jax.experimental.pallas import tpu_sc as plsc

_BATCH = 16384
_D = 64
_L = 16

_info = plsc.get_sparse_core_info()
_NC, _NS = _info.num_cores, _info.num_subcores
_NW = _NC * _NS
_BPW = _BATCH // _NW
_CHUNKS = _D // _L


@functools.partial(
    pl.kernel,
    mesh=plsc.VectorSubcoreMesh(core_axis_name="c", subcore_axis_name="s"),
    out_type=jax.ShapeDtypeStruct((_NW, _L), jnp.float32),
    scratch_types=[
        pltpu.VMEM((_BPW,), jnp.int32),
        pltpu.VMEM((_BPW, _D), jnp.float32),
        pltpu.VMEM((_BPW, _D), jnp.float32),
        pltpu.VMEM((_L,), jnp.float32),
        pltpu.SemaphoreType.DMA,
        pltpu.SemaphoreType.DMA,
    ],
    compiler_params=pltpu.CompilerParams(use_tc_tiling_on_sc=False),
)
def _center_loss_sc(features_hbm, labels_hbm, centers_hbm, out_hbm,
                    idx_v, feat_v, rows_v, acc_v, gsem, fsem):
    wid = lax.axis_index("s") * _NC + lax.axis_index("c")
    base = wid * _BPW

    fcopy = pltpu.async_copy(features_hbm.at[pl.ds(base, _BPW)], feat_v, fsem)
    pltpu.sync_copy(labels_hbm.at[pl.ds(base, _BPW)], idx_v)
    pltpu.async_copy(centers_hbm.at[idx_v], rows_v, gsem).wait()
    fcopy.wait()

    zero = jnp.zeros((_L,), jnp.float32)

    def body(i, accs):
        out = []
        for c in range(_CHUNKS):
            d = feat_v[i, pl.ds(c * _L, _L)] - rows_v[i, pl.ds(c * _L, _L)]
            out.append(accs[c] + d * d)
        return tuple(out)

    accs = lax.fori_loop(0, _BPW, body, (zero,) * _CHUNKS)
    acc_v[...] = accs[0] + accs[1] + accs[2] + accs[3]
    pltpu.sync_copy(acc_v, out_hbm.at[wid])


def kernel(features, labels, centers):
    partials = _center_loss_sc(features, labels.astype(jnp.int32), centers)
    return jnp.sum(partials) * (0.5 / _BATCH)
